# Initial kernel scaffold; baseline (speedup 1.0000x reference)
import jax
import jax.numpy as jnp
from jax.experimental import pallas as pl


def kernel(graph, features, W_in, b_in, W1, W2, W_out, b_out):
    raise NotImplementedError("write your pallas kernel here")


# speedup vs baseline: 10.7874x; 10.7874x over previous
import functools

import jax
import jax.numpy as jnp
from jax import lax
from jax.experimental import pallas as pl
from jax.experimental.pallas import tpu as pltpu
from jax.experimental.pallas import tpu_sc as plsc

N = 10000
E = 320000
F_IN = 128
H = 128
C = 64
ALPHA = 1.0

_NC = 2
_NS = 16
_NW = _NC * _NS
_EPW = E // _NW
_CH = 80
_NCH = _EPW // _CH
_RPT = 640
_ZR = 128

_NP = 10240

_mesh = plsc.VectorSubcoreMesh(core_axis_name="c", subcore_axis_name="s")


@functools.partial(
    pl.kernel,
    out_type=jax.ShapeDtypeStruct((_NC, _NP), jnp.float32),
    mesh=_mesh,
    scratch_types=[
        pltpu.VMEM((_CH,), jnp.int32),
        pltpu.VMEM((_CH,), jnp.float32),
        pltpu.VMEM((640,), jnp.float32),
        pltpu.VMEM_SHARED((_NP,), jnp.float32),
    ],
)
def _sc_degree(dst_hbm, out_hbm, dst_v, ones_v, zbuf, deg_sh):
    cid = lax.axis_index("c")
    sid = lax.axis_index("s")
    wid = sid * _NC + cid

    def _zvec(i, _):
        zbuf[pl.ds(i * 16, 16)] = jnp.zeros((16,), jnp.float32)
        return 0
    lax.fori_loop(0, 40, _zvec, 0)

    def _ovec(i, _):
        ones_v[pl.ds(i * 16, 16)] = jnp.ones((16,), jnp.float32)
        return 0
    lax.fori_loop(0, _CH // 16, _ovec, 0)

    pltpu.sync_copy(zbuf, deg_sh.at[pl.ds(sid * 640, 640)])

    plsc.subcore_barrier()

    base = wid * _EPW

    def _chunk(ch, _):
        pltpu.sync_copy(dst_hbm.at[pl.ds(base + ch * _CH, _CH)], dst_v)
        pltpu.sync_copy(ones_v, deg_sh.at[dst_v], add=True)
        return 0
    lax.fori_loop(0, _NCH, _chunk, 0)

    plsc.subcore_barrier()

    pltpu.sync_copy(deg_sh.at[pl.ds(sid * 640, 640)],
                    out_hbm.at[cid, pl.ds(sid * 640, 640)])


@functools.partial(
    pl.kernel,
    out_type=jax.ShapeDtypeStruct((_NC, _NP, H), jnp.float32),
    mesh=_mesh,
    scratch_types=[
        pltpu.VMEM((_CH,), jnp.int32),
        pltpu.VMEM((_CH,), jnp.int32),
        pltpu.VMEM((_CH, H), jnp.float32),
        pltpu.VMEM((_ZR, H), jnp.float32),
        pltpu.VMEM_SHARED((_NP, H), jnp.float32),
        pltpu.SemaphoreType.DMA,
    ],
)
def _sc_scatter(hp_hbm, src_hbm, dst_hbm, out_hbm,
                src_v, dst_v, rows_v, zbuf, agg_sh, sem):
    cid = lax.axis_index("c")
    sid = lax.axis_index("s")
    wid = sid * _NC + cid

    def _zvec(i, _):
        zbuf[i // 8, pl.ds((i % 8) * 16, 16)] = jnp.zeros((16,), jnp.float32)
        return 0
    lax.fori_loop(0, _ZR * (H // 16), _zvec, 0)

    def _zchunk(k, _):
        pltpu.sync_copy(zbuf, agg_sh.at[pl.ds(sid * _RPT + k * _ZR, _ZR)])
        return 0
    lax.fori_loop(0, _RPT // _ZR, _zchunk, 0)

    plsc.subcore_barrier()

    base = wid * _EPW

    def _chunk(ch, _):
        off = base + ch * _CH
        pltpu.sync_copy(src_hbm.at[pl.ds(off, _CH)], src_v)
        pltpu.sync_copy(dst_hbm.at[pl.ds(off, _CH)], dst_v)
        pltpu.async_copy(hp_hbm.at[src_v], rows_v, sem).wait()
        pltpu.sync_copy(rows_v, agg_sh.at[dst_v], add=True)
        return 0
    lax.fori_loop(0, _NCH, _chunk, 0)

    plsc.subcore_barrier()

    def _wchunk(k, _):
        r0 = sid * _RPT + k * _ZR
        pltpu.sync_copy(agg_sh.at[pl.ds(r0, _ZR)],
                        out_hbm.at[cid, pl.ds(r0, _ZR)])
        return 0
    lax.fori_loop(0, _RPT // _ZR, _wchunk, 0)


_BN = 1000
_GRID = N // _BN


def _tc_in_body(feat_ref, w_ref, b_ref, d0_ref, d1_ref,
                h0_ref, h0p_ref, dis_ref):
    x = feat_ref[...]
    h0 = jnp.maximum(
        jnp.dot(x, w_ref[...], preferred_element_type=jnp.float32)
        + b_ref[...], 0.0)
    s = d0_ref[...] + d1_ref[...]
    dis = lax.rsqrt(jnp.maximum(s, 1.0))
    h0_ref[...] = h0
    h0p_ref[...] = h0 * dis
    dis_ref[...] = dis


def _tc_in(features, w_in, b_in, d0, d1):
    return pl.pallas_call(
        _tc_in_body,
        grid=(_GRID,),
        in_specs=[
            pl.BlockSpec((_BN, F_IN), lambda i: (i, 0)),
            pl.BlockSpec((F_IN, H), lambda i: (0, 0)),
            pl.BlockSpec((1, H), lambda i: (0, 0)),
            pl.BlockSpec((_BN, 1), lambda i: (i, 0)),
            pl.BlockSpec((_BN, 1), lambda i: (i, 0)),
        ],
        out_specs=[
            pl.BlockSpec((_BN, H), lambda i: (i, 0)),
            pl.BlockSpec((_BN, H), lambda i: (i, 0)),
            pl.BlockSpec((_BN, 1), lambda i: (i, 0)),
        ],
        out_shape=[
            jax.ShapeDtypeStruct((N, H), jnp.float32),
            jax.ShapeDtypeStruct((N, H), jnp.float32),
            jax.ShapeDtypeStruct((N, 1), jnp.float32),
        ],
    )(features, w_in, b_in, d0, d1)


def _tc_layer_body(raw_ref, dis_ref, h0_ref, w_ref, hp_ref):
    dis = dis_ref[...]
    t = (raw_ref[0] + raw_ref[1]) * dis + ALPHA * h0_ref[...]
    h = jnp.maximum(
        jnp.dot(t, w_ref[...], preferred_element_type=jnp.float32), 0.0)
    hp_ref[...] = h * dis


def _tc_layer(raw, dis, h0, w):
    return pl.pallas_call(
        _tc_layer_body,
        grid=(_GRID,),
        in_specs=[
            pl.BlockSpec((_NC, _BN, H), lambda i: (0, i, 0)),
            pl.BlockSpec((_BN, 1), lambda i: (i, 0)),
            pl.BlockSpec((_BN, H), lambda i: (i, 0)),
            pl.BlockSpec((H, H), lambda i: (0, 0)),
        ],
        out_specs=pl.BlockSpec((_BN, H), lambda i: (i, 0)),
        out_shape=jax.ShapeDtypeStruct((N, H), jnp.float32),
    )(raw, dis, h0, w)


def _tc_out_body(raw_ref, dis_ref, h0_ref, w_ref, wo_ref, bo_ref, out_ref):
    t = (raw_ref[0] + raw_ref[1]) * dis_ref[...] + ALPHA * h0_ref[...]
    h = jnp.maximum(
        jnp.dot(t, w_ref[...], preferred_element_type=jnp.float32), 0.0)
    out_ref[...] = (
        jnp.dot(h, wo_ref[...], preferred_element_type=jnp.float32)
        + bo_ref[...])


def _tc_out(raw, dis, h0, w, w_out, b_out):
    return pl.pallas_call(
        _tc_out_body,
        grid=(_GRID,),
        in_specs=[
            pl.BlockSpec((_NC, _BN, H), lambda i: (0, i, 0)),
            pl.BlockSpec((_BN, 1), lambda i: (i, 0)),
            pl.BlockSpec((_BN, H), lambda i: (i, 0)),
            pl.BlockSpec((H, H), lambda i: (0, 0)),
            pl.BlockSpec((H, C), lambda i: (0, 0)),
            pl.BlockSpec((1, C), lambda i: (0, 0)),
        ],
        out_specs=pl.BlockSpec((_BN, C), lambda i: (i, 0)),
        out_shape=jax.ShapeDtypeStruct((N, C), jnp.float32),
    )(raw, dis, h0, w, w_out, b_out)


def kernel(graph, features, W_in, b_in, W1, W2, W_out, b_out):
    src = graph[0]
    dst = graph[1]
    deg_p = _sc_degree(dst)
    d0 = deg_p[0, :N].reshape(N, 1)
    d1 = deg_p[1, :N].reshape(N, 1)
    h0, h0p, dis = _tc_in(features, W_in, b_in.reshape(1, H), d0, d1)
    raw1 = _sc_scatter(h0p, src, dst)
    h1p = _tc_layer(raw1, dis, h0, W1)
    raw2 = _sc_scatter(h1p, src, dst)
    return _tc_out(raw2, dis, h0, W2, W_out, b_out.reshape(1, C))
